# Initial kernel scaffold; baseline (speedup 1.0000x reference)
import jax
import jax.numpy as jnp
from jax.experimental import pallas as pl


def kernel(inputs, targets):
    raise NotImplementedError("write your pallas kernel here")


# speedup vs baseline: 4.9173x; 4.9173x over previous
import functools

import jax
import jax.numpy as jnp
from jax.experimental import pallas as pl

RATIO = 0.2


def _acc_kernel(x_ref, t_ref, s1_ref, s2_ref, hist_ref):
    x = x_ref[0]
    t = t_ref[0]
    C = x.shape[0]

    m = jnp.max(x, axis=0)
    e = jnp.exp(x - m[None])
    lse = m + jnp.log(jnp.sum(e, axis=0))

    cidx = jax.lax.broadcasted_iota(jnp.int32, x.shape, 0)
    amax = jnp.min(jnp.where(x == m[None], cidx, C), axis=0)
    onehot = (cidx == amax[None]).astype(jnp.float32)

    s1 = jnp.sum(t * x, axis=(1, 2))[None, None]
    s2 = jnp.sum(t * lse[None], axis=(1, 2))[None, None]
    h = jnp.sum(onehot, axis=(1, 2))[None, None]

    @pl.when(pl.program_id(1) == 0)
    def _init():
        s1_ref[...] = s1
        s2_ref[...] = s2
        hist_ref[...] = h

    @pl.when(pl.program_id(1) != 0)
    def _acc():
        s1_ref[...] += s1
        s2_ref[...] += s2
        hist_ref[...] += h


def _combine_kernel(s1_ref, s2_ref, hist_ref, out_ref, *, denom):
    hist = hist_ref[...]
    hist = jnp.where(hist == 0.0, 1.0, hist)
    total = jnp.sum(hist, axis=2, keepdims=True)
    w = jnp.exp(RATIO * (jnp.log(total) - jnp.log(hist)))
    loss = jnp.sum(w * (s1_ref[...] - s2_ref[...]))
    out_ref[...] = jnp.full((1, 1), -loss / denom, jnp.float32)


@jax.jit
def kernel(inputs, targets):
    N, C, H, W = inputs.shape
    Hb = 24
    grid = (N, H // Hb)

    big_spec = pl.BlockSpec((1, C, Hb, W), lambda n, h: (n, 0, h, 0))
    acc_spec = pl.BlockSpec((1, 1, C), lambda n, h: (n, 0, 0))
    acc_shape = jax.ShapeDtypeStruct((N, 1, C), jnp.float32)

    s1, s2, hist = pl.pallas_call(
        _acc_kernel,
        grid=grid,
        in_specs=[big_spec, big_spec],
        out_specs=[acc_spec, acc_spec, acc_spec],
        out_shape=[acc_shape, acc_shape, acc_shape],
    )(inputs, targets)

    loss = pl.pallas_call(
        functools.partial(_combine_kernel, denom=float(N * H * W)),
        out_shape=jax.ShapeDtypeStruct((1, 1), jnp.float32),
    )(s1, s2, hist)
    return loss[0, 0]


# speedup vs baseline: 5.3471x; 1.0874x over previous
import functools

import jax
import jax.numpy as jnp
from jax.experimental import pallas as pl

RATIO = 0.2


def _acc_kernel(x_ref, t_ref, s1_ref, s2_ref, hist_ref):
    x = x_ref[0]
    t = t_ref[0]
    C = x.shape[0]

    m = jnp.max(x, axis=0)
    e = jnp.exp(x - m[None])
    lse = m + jnp.log(jnp.sum(e, axis=0))

    cidx = jax.lax.broadcasted_iota(jnp.int32, x.shape, 0)
    amax = jnp.min(jnp.where(x == m[None], cidx, C), axis=0)
    onehot = (cidx == amax[None]).astype(jnp.float32)

    s1 = jnp.sum(t * x, axis=(1, 2))[None, None]
    s2 = jnp.sum(t * lse[None], axis=(1, 2))[None, None]
    h = jnp.sum(onehot, axis=(1, 2))[None, None]

    @pl.when(pl.program_id(1) == 0)
    def _init():
        s1_ref[...] = s1
        s2_ref[...] = s2
        hist_ref[...] = h

    @pl.when(pl.program_id(1) != 0)
    def _acc():
        s1_ref[...] += s1
        s2_ref[...] += s2
        hist_ref[...] += h


def _combine_kernel(s1_ref, s2_ref, hist_ref, out_ref, *, denom):
    hist = hist_ref[...]
    hist = jnp.where(hist == 0.0, 1.0, hist)
    total = jnp.sum(hist, axis=2, keepdims=True)
    w = jnp.exp(RATIO * (jnp.log(total) - jnp.log(hist)))
    loss = jnp.sum(w * (s1_ref[...] - s2_ref[...]))
    out_ref[...] = jnp.full((1, 1), -loss / denom, jnp.float32)


@jax.jit
def kernel(inputs, targets):
    N, C, H, W = inputs.shape
    Hb = 48
    grid = (N, H // Hb)

    big_spec = pl.BlockSpec((1, C, Hb, W), lambda n, h: (n, 0, h, 0))
    acc_spec = pl.BlockSpec((1, 1, C), lambda n, h: (n, 0, 0))
    acc_shape = jax.ShapeDtypeStruct((N, 1, C), jnp.float32)

    s1, s2, hist = pl.pallas_call(
        _acc_kernel,
        grid=grid,
        in_specs=[big_spec, big_spec],
        out_specs=[acc_spec, acc_spec, acc_spec],
        out_shape=[acc_shape, acc_shape, acc_shape],
    )(inputs, targets)

    loss = pl.pallas_call(
        functools.partial(_combine_kernel, denom=float(N * H * W)),
        out_shape=jax.ShapeDtypeStruct((1, 1), jnp.float32),
    )(s1, s2, hist)
    return loss[0, 0]


# speedup vs baseline: 5.6504x; 1.0567x over previous
import functools

import jax
import jax.numpy as jnp
from jax.experimental import pallas as pl

RATIO = 0.2


def _acc_kernel(x_ref, t_ref, s1_ref, s2_ref, hist_ref):
    x = x_ref[0]
    t = t_ref[0]
    C = x.shape[0]

    m = jnp.max(x, axis=0)
    e = jnp.exp(x - m[None])
    lse = m + jnp.log(jnp.sum(e, axis=0))

    cidx = jax.lax.broadcasted_iota(jnp.int32, x.shape, 0)
    amax = jnp.min(jnp.where(x == m[None], cidx, C), axis=0)
    onehot = (cidx == amax[None]).astype(jnp.float32)

    s1 = jnp.sum(t * x, axis=(1, 2))[None, None]
    s2 = jnp.sum(t * lse[None], axis=(1, 2))[None, None]
    h = jnp.sum(onehot, axis=(1, 2))[None, None]

    @pl.when(pl.program_id(1) == 0)
    def _init():
        s1_ref[...] = s1
        s2_ref[...] = s2
        hist_ref[...] = h

    @pl.when(pl.program_id(1) != 0)
    def _acc():
        s1_ref[...] += s1
        s2_ref[...] += s2
        hist_ref[...] += h


def _combine_kernel(s1_ref, s2_ref, hist_ref, out_ref, *, denom):
    hist = hist_ref[...]
    hist = jnp.where(hist == 0.0, 1.0, hist)
    total = jnp.sum(hist, axis=2, keepdims=True)
    w = jnp.exp(RATIO * (jnp.log(total) - jnp.log(hist)))
    loss = jnp.sum(w * (s1_ref[...] - s2_ref[...]))
    out_ref[...] = jnp.full((1, 1), -loss / denom, jnp.float32)


@jax.jit
def kernel(inputs, targets):
    N, C, H, W = inputs.shape
    Hb = 64
    grid = (N, H // Hb)

    big_spec = pl.BlockSpec((1, C, Hb, W), lambda n, h: (n, 0, h, 0))
    acc_spec = pl.BlockSpec((1, 1, C), lambda n, h: (n, 0, 0))
    acc_shape = jax.ShapeDtypeStruct((N, 1, C), jnp.float32)

    s1, s2, hist = pl.pallas_call(
        _acc_kernel,
        grid=grid,
        in_specs=[big_spec, big_spec],
        out_specs=[acc_spec, acc_spec, acc_spec],
        out_shape=[acc_shape, acc_shape, acc_shape],
    )(inputs, targets)

    loss = pl.pallas_call(
        functools.partial(_combine_kernel, denom=float(N * H * W)),
        out_shape=jax.ShapeDtypeStruct((1, 1), jnp.float32),
    )(s1, s2, hist)
    return loss[0, 0]


# speedup vs baseline: 5.6506x; 1.0000x over previous
import functools

import jax
import jax.numpy as jnp
from jax.experimental import pallas as pl
from jax.experimental.pallas import tpu as pltpu

RATIO = 0.2


def _acc_kernel(x_ref, t_ref, s1_ref, s2_ref, hist_ref):
    x = x_ref[0]
    t = t_ref[0]
    C = x.shape[0]

    m = jnp.max(x, axis=0)
    e = jnp.exp(x - m[None])
    lse = m + jnp.log(jnp.sum(e, axis=0))

    cidx = jax.lax.broadcasted_iota(jnp.int32, x.shape, 0)
    amax = jnp.min(jnp.where(x == m[None], cidx, C), axis=0)
    onehot = (cidx == amax[None]).astype(jnp.float32)

    s1 = jnp.sum(t * x, axis=(1, 2))[None, None]
    s2 = jnp.sum(t * lse[None], axis=(1, 2))[None, None]
    h = jnp.sum(onehot, axis=(1, 2))[None, None]

    @pl.when(pl.program_id(1) == 0)
    def _init():
        s1_ref[...] = s1
        s2_ref[...] = s2
        hist_ref[...] = h

    @pl.when(pl.program_id(1) != 0)
    def _acc():
        s1_ref[...] += s1
        s2_ref[...] += s2
        hist_ref[...] += h


def _combine_kernel(s1_ref, s2_ref, hist_ref, out_ref, *, denom):
    hist = hist_ref[...]
    hist = jnp.where(hist == 0.0, 1.0, hist)
    total = jnp.sum(hist, axis=2, keepdims=True)
    w = jnp.exp(RATIO * (jnp.log(total) - jnp.log(hist)))
    loss = jnp.sum(w * (s1_ref[...] - s2_ref[...]))
    out_ref[...] = jnp.full((1, 1), -loss / denom, jnp.float32)


@jax.jit
def kernel(inputs, targets):
    N, C, H, W = inputs.shape
    Hb = 64
    grid = (N, H // Hb)

    big_spec = pl.BlockSpec((1, C, Hb, W), lambda n, h: (n, 0, h, 0))
    acc_spec = pl.BlockSpec((1, 1, C), lambda n, h: (n, 0, 0))
    acc_shape = jax.ShapeDtypeStruct((N, 1, C), jnp.float32)

    s1, s2, hist = pl.pallas_call(
        _acc_kernel,
        grid=grid,
        in_specs=[big_spec, big_spec],
        out_specs=[acc_spec, acc_spec, acc_spec],
        out_shape=[acc_shape, acc_shape, acc_shape],
        compiler_params=pltpu.CompilerParams(
            dimension_semantics=("parallel", "arbitrary")
        ),
    )(inputs, targets)

    loss = pl.pallas_call(
        functools.partial(_combine_kernel, denom=float(N * H * W)),
        out_shape=jax.ShapeDtypeStruct((1, 1), jnp.float32),
    )(s1, s2, hist)
    return loss[0, 0]
